# baseline (speedup 1.0000x reference)
import functools

import jax
import jax.numpy as jnp
from jax.experimental import pallas as pl
from jax.experimental.pallas import tpu as pltpu


def _excite(pooled, w1t_ref, b1_ref, w2t_ref, b2_ref):
    h = jnp.dot(pooled, w1t_ref[...], preferred_element_type=jnp.float32)
    h = jnp.maximum(h + b1_ref[...], 0.0)
    s = jnp.dot(h, w2t_ref[...], preferred_element_type=jnp.float32)
    return jax.nn.sigmoid(s + b2_ref[...])


def _se_manual_kernel(y_ref, w1t_ref, b1_ref, w2t_ref, b2_ref, o_ref,
                      xbuf, in_sems, out_sems, *, K, ch, nbh, C, inv_hw):
    p = pl.program_id(0)
    n0 = p * nbh

    def in_copy(k):
        return pltpu.make_async_copy(
            y_ref.at[pl.ds(k * ch, ch), pl.ds(n0, nbh), :],
            xbuf.at[pl.ds(k * ch, ch)],
            in_sems.at[k])

    def out_copy(k):
        return pltpu.make_async_copy(
            xbuf.at[pl.ds(k * ch, ch)],
            o_ref.at[pl.ds(k * ch, ch), pl.ds(n0, nbh), :],
            out_sems.at[k])

    for k in range(K):
        in_copy(k).start()

    acc = jnp.zeros((nbh, C), jnp.float32)
    for k in range(K):
        in_copy(k).wait()
        acc = acc + jnp.sum(xbuf[k * ch:(k + 1) * ch], axis=0)

    gate = _excite(acc * inv_hw, w1t_ref, b1_ref, w2t_ref, b2_ref)

    for k in range(K):
        xbuf[k * ch:(k + 1) * ch] = (
            xbuf[k * ch:(k + 1) * ch] * gate[None, :, :])
        out_copy(k).start()
    for k in range(K):
        out_copy(k).wait()


def _se_native_kernel(x_ref, w1t_ref, b1_ref, w2t_ref, b2_ref, o_ref,
                      *, inv_hw):
    pooled = jnp.sum(x_ref[...], axis=-1, dtype=jnp.float32) * inv_hw
    gate = _excite(pooled, w1t_ref, b1_ref, w2t_ref, b2_ref)
    o_ref[...] = (x_ref[...] * gate[:, :, None]).astype(o_ref.dtype)


def _pick_chunk(HW, bytes_per_row, budget_bytes):
    best = None
    for ch in range(1, HW + 1):
        if HW % ch or HW // ch < 3:
            continue
        if ch * bytes_per_row > budget_bytes:
            continue
        best = ch
    return best


def kernel(x, w1, b1, w2, b2):
    N, C, H, W = x.shape
    Cr = w1.shape[0]
    HW = H * W
    itemsize = jnp.dtype(x.dtype).itemsize

    w1t = jnp.transpose(w1.reshape(Cr, C))
    w2t = jnp.transpose(w2.reshape(C, Cr))
    b1r = b1.reshape(1, Cr)
    b2r = b2.reshape(1, C)
    w_specs = [
        pl.BlockSpec((C, Cr), lambda n: (0, 0)),
        pl.BlockSpec((1, Cr), lambda n: (0, 0)),
        pl.BlockSpec((Cr, C), lambda n: (0, 0)),
        pl.BlockSpec((1, C), lambda n: (0, 0)),
    ]

    nbh = N // 2
    slab_bytes = HW * nbh * C * itemsize
    ch = _pick_chunk(HW, nbh * C * itemsize, 8 << 20) if N % 16 == 0 else None

    if ch is not None and C % 128 == 0 and slab_bytes <= (52 << 20):
        y = jnp.transpose(x.reshape(N, C, HW), (2, 0, 1))
        K = HW // ch

        out_t = pl.pallas_call(
            functools.partial(_se_manual_kernel, K=K, ch=ch, nbh=nbh, C=C,
                              inv_hw=1.0 / HW),
            out_shape=jax.ShapeDtypeStruct((HW, N, C), x.dtype),
            grid_spec=pl.GridSpec(
                grid=(2,),
                in_specs=[pl.BlockSpec(memory_space=pl.ANY)] + w_specs,
                out_specs=pl.BlockSpec(memory_space=pl.ANY),
                scratch_shapes=[
                    pltpu.VMEM((HW, nbh, C), x.dtype),
                    pltpu.SemaphoreType.DMA((K,)),
                    pltpu.SemaphoreType.DMA((K,)),
                ],
            ),
            compiler_params=pltpu.CompilerParams(
                dimension_semantics=("parallel",),
                vmem_limit_bytes=min(slab_bytes + (8 << 20), 60 << 20),
            ),
        )(y, w1t, b1r, w2t, b2r)
        return out_t.transpose(1, 2, 0).reshape(N, C, H, W)

    x3 = x.reshape(N, C, HW)
    hw_pad = ((HW + 127) // 128) * 128
    per_batch = 4 * C * hw_pad * itemsize
    cap = max(1, (24 << 20) // per_batch)
    nb = 1
    for cand in range(1, N + 1):
        if N % cand == 0 and cand <= cap and (N // cand >= 4 or N < 4):
            nb = cand
    grid = (N // nb,)

    x_spec = pl.BlockSpec((nb, C, HW), lambda n: (n, 0, 0))
    out_flat = pl.pallas_call(
        functools.partial(_se_native_kernel, inv_hw=1.0 / HW),
        out_shape=jax.ShapeDtypeStruct((N, C, HW), x.dtype),
        grid_spec=pl.GridSpec(
            grid=grid,
            in_specs=[x_spec] + w_specs,
            out_specs=x_spec,
        ),
        compiler_params=pltpu.CompilerParams(
            dimension_semantics=("parallel",),
            vmem_limit_bytes=min(nb * per_batch + (8 << 20), 48 << 20),
        ),
    )(x3, w1t, b1r, w2t, b2r)
    return out_flat.reshape(N, C, H, W)


# speedup vs baseline: 1.2116x; 1.2116x over previous
import functools

import jax
import jax.numpy as jnp
from jax.experimental import pallas as pl
from jax.experimental.pallas import tpu as pltpu


def _excite(pooled, w1t_ref, b1_ref, w2t_ref, b2_ref):
    h = jnp.dot(pooled, w1t_ref[...], preferred_element_type=jnp.float32)
    h = jnp.maximum(h + b1_ref[...], 0.0)
    s = jnp.dot(h, w2t_ref[...], preferred_element_type=jnp.float32)
    return jax.nn.sigmoid(s + b2_ref[...])


def _se_manual_kernel(y_ref, w1t_ref, b1_ref, w2t_ref, b2_ref, o_ref,
                      xbuf, in_sems, out_sems, *, K, ch, nbh, C, inv_hw):
    p = pl.program_id(0)
    n0 = p * nbh

    def in_copy(k):
        return pltpu.make_async_copy(
            y_ref.at[pl.ds(k * ch, ch), pl.ds(n0, nbh), :],
            xbuf.at[pl.ds(k * ch, ch)],
            in_sems.at[k])

    def out_copy(k):
        return pltpu.make_async_copy(
            xbuf.at[pl.ds(k * ch, ch)],
            o_ref.at[pl.ds(k * ch, ch), pl.ds(n0, nbh), :],
            out_sems.at[k])

    for k in range(K):
        in_copy(k).start()

    acc = jnp.zeros((nbh, C), jnp.float32)
    for k in range(K):
        in_copy(k).wait()
        acc = acc + jnp.sum(xbuf[k * ch:(k + 1) * ch], axis=0)

    gate = _excite(acc * inv_hw, w1t_ref, b1_ref, w2t_ref, b2_ref)

    for k in range(K):
        xbuf[k * ch:(k + 1) * ch] = (
            xbuf[k * ch:(k + 1) * ch] * gate[None, :, :])
        out_copy(k).start()
    for k in range(K):
        out_copy(k).wait()


def _se_native_kernel(x_ref, w1t_ref, b1_ref, w2t_ref, b2_ref, o_ref,
                      *, inv_hw):
    pooled = jnp.sum(x_ref[...], axis=-1, dtype=jnp.float32) * inv_hw
    gate = _excite(pooled, w1t_ref, b1_ref, w2t_ref, b2_ref)
    o_ref[...] = (x_ref[...] * gate[:, :, None]).astype(o_ref.dtype)


def _pick_chunk(HW, bytes_per_row, budget_bytes):
    best = None
    for ch in range(1, HW + 1):
        if HW % ch or HW // ch < 3:
            continue
        if ch * bytes_per_row > budget_bytes:
            continue
        best = ch
    return best


def kernel(x, w1, b1, w2, b2):
    N, C, H, W = x.shape
    Cr = w1.shape[0]
    HW = H * W
    itemsize = jnp.dtype(x.dtype).itemsize

    w1t = jnp.transpose(w1.reshape(Cr, C))
    w2t = jnp.transpose(w2.reshape(C, Cr))
    b1r = b1.reshape(1, Cr)
    b2r = b2.reshape(1, C)
    w_specs = [
        pl.BlockSpec((C, Cr), lambda n: (0, 0)),
        pl.BlockSpec((1, Cr), lambda n: (0, 0)),
        pl.BlockSpec((Cr, C), lambda n: (0, 0)),
        pl.BlockSpec((1, C), lambda n: (0, 0)),
    ]

    nbh = N
    slab_bytes = HW * nbh * C * itemsize
    ch = _pick_chunk(HW, nbh * C * itemsize, 8 << 20) if N % 8 == 0 else None

    if ch is not None and C % 128 == 0 and slab_bytes <= (52 << 20):
        y = jnp.transpose(x.reshape(N, C, HW), (2, 0, 1))
        K = HW // ch

        out_t = pl.pallas_call(
            functools.partial(_se_manual_kernel, K=K, ch=ch, nbh=nbh, C=C,
                              inv_hw=1.0 / HW),
            out_shape=jax.ShapeDtypeStruct((HW, N, C), x.dtype),
            grid_spec=pl.GridSpec(
                grid=(1,),
                in_specs=[pl.BlockSpec(memory_space=pl.ANY)] + w_specs,
                out_specs=pl.BlockSpec(memory_space=pl.ANY),
                scratch_shapes=[
                    pltpu.VMEM((HW, nbh, C), x.dtype),
                    pltpu.SemaphoreType.DMA((K,)),
                    pltpu.SemaphoreType.DMA((K,)),
                ],
            ),
            compiler_params=pltpu.CompilerParams(
                dimension_semantics=("arbitrary",),
                vmem_limit_bytes=min(slab_bytes + (8 << 20), 60 << 20),
            ),
        )(y, w1t, b1r, w2t, b2r)
        return out_t.transpose(1, 2, 0).reshape(N, C, H, W)

    x3 = x.reshape(N, C, HW)
    hw_pad = ((HW + 127) // 128) * 128
    per_batch = 4 * C * hw_pad * itemsize
    cap = max(1, (24 << 20) // per_batch)
    nb = 1
    for cand in range(1, N + 1):
        if N % cand == 0 and cand <= cap and (N // cand >= 4 or N < 4):
            nb = cand
    grid = (N // nb,)

    x_spec = pl.BlockSpec((nb, C, HW), lambda n: (n, 0, 0))
    out_flat = pl.pallas_call(
        functools.partial(_se_native_kernel, inv_hw=1.0 / HW),
        out_shape=jax.ShapeDtypeStruct((N, C, HW), x.dtype),
        grid_spec=pl.GridSpec(
            grid=grid,
            in_specs=[x_spec] + w_specs,
            out_specs=x_spec,
        ),
        compiler_params=pltpu.CompilerParams(
            dimension_semantics=("parallel",),
            vmem_limit_bytes=min(nb * per_batch + (8 << 20), 48 << 20),
        ),
    )(x3, w1t, b1r, w2t, b2r)
    return out_flat.reshape(N, C, H, W)


# speedup vs baseline: 1.3574x; 1.1203x over previous
import functools

import jax
import jax.numpy as jnp
from jax.experimental import pallas as pl
from jax.experimental.pallas import tpu as pltpu


def _excite(pooled, w1t_ref, b1_ref, w2t_ref, b2_ref):
    h = jnp.dot(pooled, w1t_ref[...], preferred_element_type=jnp.float32)
    h = jnp.maximum(h + b1_ref[...], 0.0)
    s = jnp.dot(h, w2t_ref[...], preferred_element_type=jnp.float32)
    return jax.nn.sigmoid(s + b2_ref[...])


def _se_manual_kernel(y_ref, w1t_ref, b1_ref, w2t_ref, b2_ref, o_ref,
                      xbuf, in_sems, out_sems, *, K, nch, HW, C, inv_hw):
    def in_copy(k):
        return pltpu.make_async_copy(
            y_ref.at[:, pl.ds(k * nch, nch), :],
            xbuf.at[k],
            in_sems.at[k])

    def out_copy(k):
        return pltpu.make_async_copy(
            xbuf.at[k],
            o_ref.at[:, pl.ds(k * nch, nch), :],
            out_sems.at[k])

    for k in range(K):
        in_copy(k).start()

    for k in range(K):
        in_copy(k).wait()
        pooled = jnp.sum(xbuf[k], axis=0) * inv_hw
        gate = _excite(pooled, w1t_ref, b1_ref, w2t_ref, b2_ref)
        xbuf[k] = xbuf[k] * gate[None, :, :]
        out_copy(k).start()
    for k in range(K):
        out_copy(k).wait()


def _se_native_kernel(x_ref, w1t_ref, b1_ref, w2t_ref, b2_ref, o_ref,
                      *, inv_hw):
    pooled = jnp.sum(x_ref[...], axis=-1, dtype=jnp.float32) * inv_hw
    gate = _excite(pooled, w1t_ref, b1_ref, w2t_ref, b2_ref)
    o_ref[...] = (x_ref[...] * gate[:, :, None]).astype(o_ref.dtype)


def _pick_chunk(HW, bytes_per_row, budget_bytes):
    best = None
    for ch in range(1, HW + 1):
        if HW % ch or HW // ch < 3:
            continue
        if ch * bytes_per_row > budget_bytes:
            continue
        best = ch
    return best


def kernel(x, w1, b1, w2, b2):
    N, C, H, W = x.shape
    Cr = w1.shape[0]
    HW = H * W
    itemsize = jnp.dtype(x.dtype).itemsize

    w1t = jnp.transpose(w1.reshape(Cr, C))
    w2t = jnp.transpose(w2.reshape(C, Cr))
    b1r = b1.reshape(1, Cr)
    b2r = b2.reshape(1, C)
    w_specs = [
        pl.BlockSpec((C, Cr), lambda n: (0, 0)),
        pl.BlockSpec((1, Cr), lambda n: (0, 0)),
        pl.BlockSpec((Cr, C), lambda n: (0, 0)),
        pl.BlockSpec((1, C), lambda n: (0, 0)),
    ]

    nch = 64 if N % 64 == 0 and N > 64 else N
    K = N // nch
    slab_bytes = HW * N * C * itemsize

    if (N % 8 == 0 and C % 128 == 0 and nch % 8 == 0
            and slab_bytes <= (52 << 20)):
        y = jnp.transpose(x.reshape(N, C, HW), (2, 0, 1))

        out_t = pl.pallas_call(
            functools.partial(_se_manual_kernel, K=K, nch=nch, HW=HW, C=C,
                              inv_hw=1.0 / HW),
            out_shape=jax.ShapeDtypeStruct((HW, N, C), x.dtype),
            grid_spec=pl.GridSpec(
                grid=(1,),
                in_specs=[pl.BlockSpec(memory_space=pl.ANY)] + w_specs,
                out_specs=pl.BlockSpec(memory_space=pl.ANY),
                scratch_shapes=[
                    pltpu.VMEM((K, HW, nch, C), x.dtype),
                    pltpu.SemaphoreType.DMA((K,)),
                    pltpu.SemaphoreType.DMA((K,)),
                ],
            ),
            compiler_params=pltpu.CompilerParams(
                dimension_semantics=("arbitrary",),
                vmem_limit_bytes=min(slab_bytes + (8 << 20), 60 << 20),
            ),
        )(y, w1t, b1r, w2t, b2r)
        return out_t.transpose(1, 2, 0).reshape(N, C, H, W)

    x3 = x.reshape(N, C, HW)
    hw_pad = ((HW + 127) // 128) * 128
    per_batch = 4 * C * hw_pad * itemsize
    cap = max(1, (24 << 20) // per_batch)
    nb = 1
    for cand in range(1, N + 1):
        if N % cand == 0 and cand <= cap and (N // cand >= 4 or N < 4):
            nb = cand
    grid = (N // nb,)

    x_spec = pl.BlockSpec((nb, C, HW), lambda n: (n, 0, 0))
    out_flat = pl.pallas_call(
        functools.partial(_se_native_kernel, inv_hw=1.0 / HW),
        out_shape=jax.ShapeDtypeStruct((N, C, HW), x.dtype),
        grid_spec=pl.GridSpec(
            grid=grid,
            in_specs=[x_spec] + w_specs,
            out_specs=x_spec,
        ),
        compiler_params=pltpu.CompilerParams(
            dimension_semantics=("parallel",),
            vmem_limit_bytes=min(nb * per_batch + (8 << 20), 48 << 20),
        ),
    )(x3, w1t, b1r, w2t, b2r)
    return out_flat.reshape(N, C, H, W)


# speedup vs baseline: 1.3794x; 1.0162x over previous
import functools

import jax
import jax.numpy as jnp
from jax.experimental import pallas as pl
from jax.experimental.pallas import tpu as pltpu


def _se_hwnc_kernel(x_ref, w1r_ref, b1_ref, w2t_ref, b2_ref, o_ref, *, inv_hw):
    pooled = jnp.sum(x_ref[...], axis=0) * inv_hw

    h = jax.lax.dot_general(pooled, w1r_ref[...], (((1,), (1,)), ((), ())),
                            preferred_element_type=jnp.float32)
    h = jnp.maximum(h + b1_ref[...], 0.0)
    s = jnp.dot(h, w2t_ref[...], preferred_element_type=jnp.float32)
    gate = jax.nn.sigmoid(s + b2_ref[...])

    o_ref[...] = (x_ref[...] * gate[None, :, :]).astype(o_ref.dtype)


def _se_native_kernel(x_ref, w1t_ref, b1_ref, w2t_ref, b2_ref, o_ref,
                      *, inv_hw):
    pooled = jnp.sum(x_ref[...], axis=-1, dtype=jnp.float32) * inv_hw
    h = jnp.dot(pooled, w1t_ref[...], preferred_element_type=jnp.float32)
    h = jnp.maximum(h + b1_ref[...], 0.0)
    s = jnp.dot(h, w2t_ref[...], preferred_element_type=jnp.float32)
    gate = jax.nn.sigmoid(s + b2_ref[...])
    o_ref[...] = (x_ref[...] * gate[:, :, None]).astype(o_ref.dtype)


def _pick_nb(N, per_batch_bytes, budget_bytes, min_steps):
    cap = max(1, budget_bytes // per_batch_bytes)
    best = 1
    for nb in range(1, N + 1):
        if N % nb or nb > cap:
            continue
        if N // nb < min_steps and N >= min_steps:
            continue
        best = nb
    return best


def kernel(x, w1, b1, w2, b2):
    N, C, H, W = x.shape
    Cr = w1.shape[0]
    HW = H * W
    itemsize = jnp.dtype(x.dtype).itemsize

    w1r = w1.reshape(Cr, C)
    w2t = jnp.transpose(w2.reshape(C, Cr))
    b1r = b1.reshape(1, Cr)
    b2r = b2.reshape(1, C)
    w_specs = [
        pl.BlockSpec((Cr, C), lambda n: (0, 0)),
        pl.BlockSpec((1, Cr), lambda n: (0, 0)),
        pl.BlockSpec((Cr, C), lambda n: (0, 0)),
        pl.BlockSpec((1, C), lambda n: (0, 0)),
    ]

    if N % 8 == 0 and C % 128 == 0:
        y = jnp.transpose(x.reshape(N, C, HW), (2, 0, 1))

        per_batch = 4 * HW * C * itemsize
        nb = _pick_nb(N, per_batch, 54 << 20, 2)
        grid = (N // nb,)

        y_spec = pl.BlockSpec((HW, nb, C), lambda n: (0, n, 0))
        out_t = pl.pallas_call(
            functools.partial(_se_hwnc_kernel, inv_hw=1.0 / HW),
            out_shape=jax.ShapeDtypeStruct((HW, N, C), x.dtype),
            grid_spec=pl.GridSpec(
                grid=grid,
                in_specs=[y_spec] + w_specs,
                out_specs=y_spec,
            ),
            compiler_params=pltpu.CompilerParams(
                dimension_semantics=("parallel",),
                vmem_limit_bytes=min(nb * per_batch + (8 << 20), 60 << 20),
            ),
        )(y, w1r, b1r, w2t, b2r)
        return out_t.transpose(1, 2, 0).reshape(N, C, H, W)

    x3 = x.reshape(N, C, HW)
    hw_pad = ((HW + 127) // 128) * 128
    per_batch = 4 * C * hw_pad * itemsize
    nb = _pick_nb(N, per_batch, 24 << 20, 4)
    grid = (N // nb,)

    x_spec = pl.BlockSpec((nb, C, HW), lambda n: (n, 0, 0))
    w_specs[0] = pl.BlockSpec((C, Cr), lambda n: (0, 0))
    out_flat = pl.pallas_call(
        functools.partial(_se_native_kernel, inv_hw=1.0 / HW),
        out_shape=jax.ShapeDtypeStruct((N, C, HW), x.dtype),
        grid_spec=pl.GridSpec(
            grid=grid,
            in_specs=[x_spec] + w_specs,
            out_specs=x_spec,
        ),
        compiler_params=pltpu.CompilerParams(
            dimension_semantics=("parallel",),
            vmem_limit_bytes=min(nb * per_batch + (8 << 20), 48 << 20),
        ),
    )(x3, jnp.transpose(w1r), b1r, w2t, b2r)
    return out_flat.reshape(N, C, H, W)
